# baseline (speedup 1.0000x reference)
import functools

import jax
import jax.numpy as jnp
from jax import lax
from jax.experimental import pallas as pl
from jax.experimental.pallas import tpu as pltpu
from jax.experimental.pallas import tpu_sc as plsc

_E = 64
_T = 8192
_D = 768
_FF = 768
_BM = 256
_MAXT = _T // _BM + _E - 1
_TPAD = _MAXT * _BM
_NW = 32
_NCH = 4
_CH = (_T // _NW) // _NCH
_RB = 128


def _router_body(x_ref, wg_ref, w_ref, pos_ref, cnt_ref):
    x = x_ref[...]
    logits = jnp.dot(x, wg_ref[...], preferred_element_type=jnp.float32)
    m = jnp.max(logits, axis=-1, keepdims=True)
    ex = jnp.exp(logits - m)
    p = ex / jnp.sum(ex, axis=-1, keepdims=True)
    pmax = jnp.max(p, axis=-1)
    col = lax.broadcasted_iota(jnp.int32, p.shape, 1)
    etok = jnp.min(jnp.where(p >= pmax[:, None], col, _E), axis=-1)
    w_ref[...] = jnp.broadcast_to(pmax[:, None], (_T, 128))

    nb = _T // _RB
    oh = (col == etok[:, None]).astype(jnp.float32)
    oh3 = oh.reshape(nb, _RB, _E)
    r = lax.broadcasted_iota(jnp.int32, (_RB, _RB), 0)
    c = lax.broadcasted_iota(jnp.int32, (_RB, _RB), 1)
    tri = (c < r).astype(jnp.float32)
    tri3 = jnp.broadcast_to(tri[None], (nb, _RB, _RB))
    within = lax.dot_general(
        tri3, oh3, (((2,), (1,)), ((0,), (0,))),
        preferred_element_type=jnp.float32)
    btot = jnp.sum(oh3, axis=1)
    rb = lax.broadcasted_iota(jnp.int32, (nb, nb), 0)
    cb = lax.broadcasted_iota(jnp.int32, (nb, nb), 1)
    trib = (cb < rb).astype(jnp.float32)
    bbase = jnp.dot(trib, btot, preferred_element_type=jnp.float32)
    counts = jnp.sum(btot, axis=0, keepdims=True)

    pe64 = jnp.floor((counts + (_BM - 1.0)) / _BM) * (_BM // 64)
    triu = (rb < cb).astype(jnp.float32)
    off = 64.0 * jnp.dot(pe64, triu, preferred_element_type=jnp.float32)

    off_tok = jnp.sum(oh * off, axis=-1)
    bbase_tok = jnp.sum(oh3 * bbase[:, None, :], axis=-1).reshape(_T)
    within_tok = jnp.sum(oh3 * within, axis=-1).reshape(_T)
    pos_ref[...] = (off_tok + bbase_tok + within_tok).astype(jnp.int32)

    cnt_ref[...] = jnp.pad(counts, ((0, 0), (0, 128 - _E))).astype(jnp.int32)


def _router(x, wg):
    return pl.pallas_call(
        _router_body,
        out_shape=(
            jax.ShapeDtypeStruct((_T, 128), jnp.float32),
            jax.ShapeDtypeStruct((_T,), jnp.int32),
            jax.ShapeDtypeStruct((1, 128), jnp.int32),
        ),
    )(x, wg)


def _sc_gather(table, idx3):
    t_rows, d = table.shape
    nw, nch, ch = idx3.shape
    bpw = nch * ch
    mesh = plsc.VectorSubcoreMesh(core_axis_name="c", subcore_axis_name="s")

    @functools.partial(
        pl.kernel,
        mesh=mesh,
        out_type=jax.ShapeDtypeStruct((nw * bpw, d), jnp.float32),
        scratch_types=[
            pltpu.VMEM((nch, ch), jnp.int32),
            pltpu.VMEM((2, ch, d), jnp.float32),
            pltpu.SemaphoreType.DMA((2,)),
        ],
    )
    def gk(table_hbm, idx_hbm, out_hbm, idx_v, rows_v, sem):
        wid = lax.axis_index("s") * 2 + lax.axis_index("c")
        pltpu.sync_copy(idx_hbm.at[wid], idx_v)
        g = [None, None]
        g[0] = pltpu.make_async_copy(
            table_hbm.at[idx_v.at[0]], rows_v.at[0], sem.at[0])
        g[0].start()
        for j in range(nch):
            cur = j % 2
            g[cur].wait()
            if j + 1 < nch:
                g[1 - cur] = pltpu.make_async_copy(
                    table_hbm.at[idx_v.at[j + 1]], rows_v.at[1 - cur],
                    sem.at[1 - cur])
                g[1 - cur].start()
            pltpu.sync_copy(rows_v.at[cur],
                            out_hbm.at[pl.ds(wid * bpw + j * ch, ch)])

    return gk(table, idx3)


def _sc_dispatch(x, w, idx3):
    t_rows, d = x.shape
    nw, nch, ch = idx3.shape
    bpw = nch * ch
    mesh = plsc.VectorSubcoreMesh(core_axis_name="c", subcore_axis_name="s")

    @functools.partial(
        pl.kernel,
        mesh=mesh,
        out_type=(
            jax.ShapeDtypeStruct((_TPAD, d), jnp.float32),
            jax.ShapeDtypeStruct((_TPAD, 128), jnp.float32),
        ),
        scratch_types=[
            pltpu.VMEM((nch, ch), jnp.int32),
            pltpu.VMEM((2, ch, d), jnp.float32),
            pltpu.VMEM((2, ch, 128), jnp.float32),
            pltpu.SemaphoreType.DMA((2, 2)),
        ],
    )
    def sk(x_hbm, w_hbm, idx_hbm, xs_hbm, ws_hbm, idx_v, rows_v, wrows_v, sem):
        wid = lax.axis_index("s") * 2 + lax.axis_index("c")
        pltpu.sync_copy(idx_hbm.at[wid], idx_v)
        pltpu.sync_copy(x_hbm.at[pl.ds(wid * bpw, ch)], rows_v.at[0])
        pltpu.sync_copy(w_hbm.at[pl.ds(wid * bpw, ch)], wrows_v.at[0])
        for j in range(nch):
            cur = j % 2
            sx = pltpu.make_async_copy(
                rows_v.at[cur], xs_hbm.at[idx_v.at[j]], sem.at[cur, 0])
            sw = pltpu.make_async_copy(
                wrows_v.at[cur], ws_hbm.at[idx_v.at[j]], sem.at[cur, 1])
            sx.start()
            sw.start()
            if j + 1 < nch:
                base = wid * bpw + (j + 1) * ch
                pltpu.sync_copy(x_hbm.at[pl.ds(base, ch)], rows_v.at[1 - cur])
                pltpu.sync_copy(w_hbm.at[pl.ds(base, ch)], wrows_v.at[1 - cur])
            sx.wait()
            sw.wait()

    return sk(x, w, idx3)


_WQ = 4
_WR = _D // _WQ


def _wdma(w1_any, w2_any, w1buf, w2buf, sems, e, slot):
    cs = []
    for c in range(_WQ):
        cs.append(pltpu.make_async_copy(
            w1_any.at[pl.ds(e, 1), pl.ds(c * _WR, _WR)],
            w1buf.at[pl.ds(slot, 1), pl.ds(c * _WR, _WR)],
            sems.at[slot, c]))
        cs.append(pltpu.make_async_copy(
            w2_any.at[pl.ds(e, 1), pl.ds(c * _WR, _WR)],
            w2buf.at[pl.ds(slot, 1), pl.ds(c * _WR, _WR)],
            sems.at[slot, _WQ + c]))
    return cs


def _mlp_body(b_ref, e_ref, vld_ref, chg_ref, slot_ref, nxt_ref, isu_ref,
              x_ref, wt_ref, w1_any, b1_ref, w2_any, b2_ref, out_ref,
              w1buf, w2buf, sems):
    i = pl.program_id(0)
    slot = slot_ref[i]

    @pl.when(i == 0)
    def _():
        for cp in _wdma(w1_any, w2_any, w1buf, w2buf, sems, e_ref[0], 0):
            cp.start()

    @pl.when(chg_ref[i] == 1)
    def _():
        for cp in _wdma(w1_any, w2_any, w1buf, w2buf, sems, e_ref[i], slot):
            cp.wait()

        @pl.when(isu_ref[i] == 1)
        def _():
            for cp in _wdma(w1_any, w2_any, w1buf, w2buf, sems,
                            nxt_ref[i], 1 - slot):
                cp.start()

    @pl.when(vld_ref[i] == 1)
    def _():
        x = x_ref[...].astype(jnp.bfloat16)
        w = wt_ref[...][:, 0:1]
        h = jnp.dot(x, w1buf[pl.ds(slot, 1)][0].astype(jnp.bfloat16),
                    preferred_element_type=jnp.float32)
        h = jnp.maximum(h + b1_ref[0], 0.0).astype(jnp.bfloat16)
        y = jnp.dot(h, w2buf[pl.ds(slot, 1)][0].astype(jnp.bfloat16),
                    preferred_element_type=jnp.float32)
        out_ref[...] = (y + b2_ref[0]) * w


def _grouped_mlp(x_sorted, w_sorted, w1, b1, w2, b2, sched):
    sched_b, sched_e, vld, chg, slot, nxt, isu = sched
    grid_spec = pltpu.PrefetchScalarGridSpec(
        num_scalar_prefetch=7,
        grid=(_MAXT,),
        in_specs=[
            pl.BlockSpec((_BM, _D), lambda i, b, e, *_: (b[i], 0)),
            pl.BlockSpec((_BM, 128), lambda i, b, e, *_: (b[i], 0)),
            pl.BlockSpec(memory_space=pl.ANY),
            pl.BlockSpec((1, 1, _FF), lambda i, b, e, *_: (e[i], 0, 0)),
            pl.BlockSpec(memory_space=pl.ANY),
            pl.BlockSpec((1, 1, _D), lambda i, b, e, *_: (e[i], 0, 0)),
        ],
        out_specs=pl.BlockSpec((_BM, _D), lambda i, b, e, *_: (b[i], 0)),
        scratch_shapes=[
            pltpu.VMEM((2, _D, _FF), jnp.float32),
            pltpu.VMEM((2, _FF, _D), jnp.float32),
            pltpu.SemaphoreType.DMA((2, 2 * _WQ)),
        ],
    )
    return pl.pallas_call(
        _mlp_body,
        grid_spec=grid_spec,
        out_shape=jax.ShapeDtypeStruct((_TPAD, _D), jnp.float32),
        compiler_params=pltpu.CompilerParams(dimension_semantics=("arbitrary",)),
    )(sched_b, sched_e, vld, chg, slot, nxt, isu,
      x_sorted, w_sorted, w1,
      b1.reshape(_E, 1, _FF), w2, b2.reshape(_E, 1, _D))


def _schedule(counts):
    i32 = jnp.int32
    ntile = (counts + _BM - 1) // _BM
    c = jnp.concatenate([jnp.zeros((1,), i32), jnp.cumsum(ntile).astype(i32)])
    total = c[_E]
    j = jnp.arange(_MAXT, dtype=i32)
    ej = jnp.sum((c[None, :] <= j[:, None]).astype(i32), axis=1) - 1
    ej = jnp.minimum(ej, _E - 1)
    ep = jnp.sum((c <= total - 1).astype(i32)) - 1
    valid = j < total
    ej = jnp.where(valid, ej, ep).astype(i32)
    bj = jnp.where(valid, j, total - 1).astype(i32)
    vld = valid.astype(i32)

    chg = jnp.concatenate(
        [jnp.ones((1,), i32), (ej[1:] != ej[:-1]).astype(i32)])
    runidx = jnp.cumsum(chg) - 1
    slot = (runidx % 2).astype(i32)
    chgpos = jnp.where(chg == 1, j, _MAXT)
    suffmin = jnp.flip(lax.cummin(jnp.flip(chgpos)))
    nc = jnp.concatenate([suffmin[1:], jnp.full((1,), _MAXT, i32)])
    isu = ((chg == 1) & (nc < _MAXT)).astype(i32)
    nxt = ej[jnp.minimum(nc, _MAXT - 1)]
    return bj, ej, vld, chg, slot, nxt, isu


def kernel(hidden_states, Wg, W1, b1, W2, b2):
    w_tok, pos, cnt_padded = _router(hidden_states, Wg)
    counts = cnt_padded[0, :_E]
    sched = _schedule(counts)

    pos3 = pos.reshape(_NW, _NCH, _CH)
    x_sorted, w_sorted = _sc_dispatch(hidden_states, w_tok, pos3)
    y_sorted = _grouped_mlp(x_sorted, w_sorted, W1, b1, W2, b2, sched)
    out = _sc_gather(y_sorted, pos3)
    return out


# speedup vs baseline: 1.0321x; 1.0321x over previous
import functools

import jax
import jax.numpy as jnp
from jax import lax
from jax.experimental import pallas as pl
from jax.experimental.pallas import tpu as pltpu
from jax.experimental.pallas import tpu_sc as plsc

_E = 64
_T = 8192
_D = 768
_FF = 768
_BM = 256
_MAXT = _T // _BM + _E - 1
_TPAD = _MAXT * _BM
_NW = 32
_NCH = 2
_CH = (_T // _NW) // _NCH
_RB = 128


def _router_body(x_ref, wg_ref, w_ref, pos_ref, cnt_ref):
    x = x_ref[...]
    logits = jnp.dot(x, wg_ref[...], preferred_element_type=jnp.float32)
    m = jnp.max(logits, axis=-1, keepdims=True)
    ex = jnp.exp(logits - m)
    p = ex / jnp.sum(ex, axis=-1, keepdims=True)
    pmax = jnp.max(p, axis=-1)
    col = lax.broadcasted_iota(jnp.int32, p.shape, 1)
    etok = jnp.min(jnp.where(p >= pmax[:, None], col, _E), axis=-1)
    w_ref[...] = jnp.broadcast_to(pmax[:, None], (_T, 128))

    nb = _T // _RB
    oh = (col == etok[:, None]).astype(jnp.float32)
    oh3 = oh.reshape(nb, _RB, _E)
    r = lax.broadcasted_iota(jnp.int32, (_RB, _RB), 0)
    c = lax.broadcasted_iota(jnp.int32, (_RB, _RB), 1)
    tri = (c < r).astype(jnp.float32)
    tri3 = jnp.broadcast_to(tri[None], (nb, _RB, _RB))
    within = lax.dot_general(
        tri3, oh3, (((2,), (1,)), ((0,), (0,))),
        preferred_element_type=jnp.float32)
    btot = jnp.sum(oh3, axis=1)
    rb = lax.broadcasted_iota(jnp.int32, (nb, nb), 0)
    cb = lax.broadcasted_iota(jnp.int32, (nb, nb), 1)
    trib = (cb < rb).astype(jnp.float32)
    bbase = jnp.dot(trib, btot, preferred_element_type=jnp.float32)
    counts = jnp.sum(btot, axis=0, keepdims=True)

    pe64 = jnp.floor((counts + (_BM - 1.0)) / _BM) * (_BM // 64)
    triu = (rb < cb).astype(jnp.float32)
    off = 64.0 * jnp.dot(pe64, triu, preferred_element_type=jnp.float32)

    off_tok = jnp.sum(oh * off, axis=-1)
    bbase_tok = jnp.sum(oh3 * bbase[:, None, :], axis=-1).reshape(_T)
    within_tok = jnp.sum(oh3 * within, axis=-1).reshape(_T)
    pos_ref[...] = (off_tok + bbase_tok + within_tok).astype(jnp.int32)

    cnt_ref[...] = jnp.pad(counts, ((0, 0), (0, 128 - _E))).astype(jnp.int32)


def _router(x, wg):
    return pl.pallas_call(
        _router_body,
        out_shape=(
            jax.ShapeDtypeStruct((_T, 128), jnp.float32),
            jax.ShapeDtypeStruct((_T,), jnp.int32),
            jax.ShapeDtypeStruct((1, 128), jnp.int32),
        ),
    )(x, wg)


def _sc_gather(table, idx3):
    t_rows, d = table.shape
    nw, nch, ch = idx3.shape
    bpw = nch * ch
    mesh = plsc.VectorSubcoreMesh(core_axis_name="c", subcore_axis_name="s")

    @functools.partial(
        pl.kernel,
        mesh=mesh,
        out_type=jax.ShapeDtypeStruct((nw * bpw, d), jnp.float32),
        scratch_types=[
            pltpu.VMEM((nch, ch), jnp.int32),
            pltpu.VMEM((ch, d), jnp.float32),
            pltpu.SemaphoreType.DMA,
        ],
    )
    def gk(table_hbm, idx_hbm, out_hbm, idx_v, rows_v, sem):
        wid = lax.axis_index("s") * 2 + lax.axis_index("c")
        pltpu.sync_copy(idx_hbm.at[wid], idx_v)
        for j in range(nch):
            pltpu.async_copy(table_hbm.at[idx_v.at[j]], rows_v, sem).wait()
            pltpu.sync_copy(rows_v, out_hbm.at[pl.ds(wid * bpw + j * ch, ch)])

    return gk(table, idx3)


def _sc_dispatch(x, w, idx3):
    t_rows, d = x.shape
    nw, nch, ch = idx3.shape
    bpw = nch * ch
    mesh = plsc.VectorSubcoreMesh(core_axis_name="c", subcore_axis_name="s")

    @functools.partial(
        pl.kernel,
        mesh=mesh,
        out_type=(
            jax.ShapeDtypeStruct((_TPAD, d), jnp.float32),
            jax.ShapeDtypeStruct((_TPAD, 128), jnp.float32),
        ),
        scratch_types=[
            pltpu.VMEM((nch, ch), jnp.int32),
            pltpu.VMEM((ch, d), jnp.float32),
            pltpu.VMEM((ch, 128), jnp.float32),
            pltpu.SemaphoreType.DMA,
        ],
    )
    def sk(x_hbm, w_hbm, idx_hbm, xs_hbm, ws_hbm, idx_v, rows_v, wrows_v, sem):
        wid = lax.axis_index("s") * 2 + lax.axis_index("c")
        pltpu.sync_copy(idx_hbm.at[wid], idx_v)
        for j in range(nch):
            base = wid * bpw + j * ch
            pltpu.sync_copy(x_hbm.at[pl.ds(base, ch)], rows_v)
            pltpu.sync_copy(w_hbm.at[pl.ds(base, ch)], wrows_v)
            pltpu.async_copy(rows_v, xs_hbm.at[idx_v.at[j]], sem).wait()
            pltpu.async_copy(wrows_v, ws_hbm.at[idx_v.at[j]], sem).wait()

    return sk(x, w, idx3)


_WQ = 4
_WR = _D // _WQ


def _wdma(w1_any, w2_any, w1buf, w2buf, sems, e, slot):
    cs = []
    for c in range(_WQ):
        cs.append(pltpu.make_async_copy(
            w1_any.at[pl.ds(e, 1), pl.ds(c * _WR, _WR)],
            w1buf.at[pl.ds(slot, 1), pl.ds(c * _WR, _WR)],
            sems.at[slot, c]))
        cs.append(pltpu.make_async_copy(
            w2_any.at[pl.ds(e, 1), pl.ds(c * _WR, _WR)],
            w2buf.at[pl.ds(slot, 1), pl.ds(c * _WR, _WR)],
            sems.at[slot, _WQ + c]))
    return cs


def _mlp_body(b_ref, e_ref, vld_ref, chg_ref, slot_ref, nxt_ref, isu_ref,
              x_ref, wt_ref, w1_any, b1_ref, w2_any, b2_ref, out_ref,
              w1buf, w2buf, sems):
    i = pl.program_id(0)
    slot = slot_ref[i]

    @pl.when(i == 0)
    def _():
        for cp in _wdma(w1_any, w2_any, w1buf, w2buf, sems, e_ref[0], 0):
            cp.start()

    @pl.when(chg_ref[i] == 1)
    def _():
        for cp in _wdma(w1_any, w2_any, w1buf, w2buf, sems, e_ref[i], slot):
            cp.wait()

        @pl.when(isu_ref[i] == 1)
        def _():
            for cp in _wdma(w1_any, w2_any, w1buf, w2buf, sems,
                            nxt_ref[i], 1 - slot):
                cp.start()

    @pl.when(vld_ref[i] == 1)
    def _():
        x = x_ref[...].astype(jnp.bfloat16)
        w = wt_ref[...][:, 0:1]
        h = jnp.dot(x, w1buf[pl.ds(slot, 1)][0].astype(jnp.bfloat16),
                    preferred_element_type=jnp.float32)
        h = jnp.maximum(h + b1_ref[0], 0.0).astype(jnp.bfloat16)
        y = jnp.dot(h, w2buf[pl.ds(slot, 1)][0].astype(jnp.bfloat16),
                    preferred_element_type=jnp.float32)
        out_ref[...] = (y + b2_ref[0]) * w


def _grouped_mlp(x_sorted, w_sorted, w1, b1, w2, b2, sched):
    sched_b, sched_e, vld, chg, slot, nxt, isu = sched
    grid_spec = pltpu.PrefetchScalarGridSpec(
        num_scalar_prefetch=7,
        grid=(_MAXT,),
        in_specs=[
            pl.BlockSpec((_BM, _D), lambda i, b, e, *_: (b[i], 0)),
            pl.BlockSpec((_BM, 128), lambda i, b, e, *_: (b[i], 0)),
            pl.BlockSpec(memory_space=pl.ANY),
            pl.BlockSpec((1, 1, _FF), lambda i, b, e, *_: (e[i], 0, 0)),
            pl.BlockSpec(memory_space=pl.ANY),
            pl.BlockSpec((1, 1, _D), lambda i, b, e, *_: (e[i], 0, 0)),
        ],
        out_specs=pl.BlockSpec((_BM, _D), lambda i, b, e, *_: (b[i], 0)),
        scratch_shapes=[
            pltpu.VMEM((2, _D, _FF), jnp.float32),
            pltpu.VMEM((2, _FF, _D), jnp.float32),
            pltpu.SemaphoreType.DMA((2, 2 * _WQ)),
        ],
    )
    return pl.pallas_call(
        _mlp_body,
        grid_spec=grid_spec,
        out_shape=jax.ShapeDtypeStruct((_TPAD, _D), jnp.float32),
        compiler_params=pltpu.CompilerParams(dimension_semantics=("arbitrary",)),
    )(sched_b, sched_e, vld, chg, slot, nxt, isu,
      x_sorted, w_sorted, w1,
      b1.reshape(_E, 1, _FF), w2, b2.reshape(_E, 1, _D))


def _schedule(counts):
    i32 = jnp.int32
    ntile = (counts + _BM - 1) // _BM
    c = jnp.concatenate([jnp.zeros((1,), i32), jnp.cumsum(ntile).astype(i32)])
    total = c[_E]
    j = jnp.arange(_MAXT, dtype=i32)
    ej = jnp.sum((c[None, :] <= j[:, None]).astype(i32), axis=1) - 1
    ej = jnp.minimum(ej, _E - 1)
    ep = jnp.sum((c <= total - 1).astype(i32)) - 1
    valid = j < total
    ej = jnp.where(valid, ej, ep).astype(i32)
    bj = jnp.where(valid, j, total - 1).astype(i32)
    vld = valid.astype(i32)

    chg = jnp.concatenate(
        [jnp.ones((1,), i32), (ej[1:] != ej[:-1]).astype(i32)])
    runidx = jnp.cumsum(chg) - 1
    slot = (runidx % 2).astype(i32)
    chgpos = jnp.where(chg == 1, j, _MAXT)
    suffmin = jnp.flip(lax.cummin(jnp.flip(chgpos)))
    nc = jnp.concatenate([suffmin[1:], jnp.full((1,), _MAXT, i32)])
    isu = ((chg == 1) & (nc < _MAXT)).astype(i32)
    nxt = ej[jnp.minimum(nc, _MAXT - 1)]
    return bj, ej, vld, chg, slot, nxt, isu


def kernel(hidden_states, Wg, W1, b1, W2, b2):
    w_tok, pos, cnt_padded = _router(hidden_states, Wg)
    counts = cnt_padded[0, :_E]
    sched = _schedule(counts)

    pos3 = pos.reshape(_NW, _NCH, _CH)
    x_sorted, w_sorted = _sc_dispatch(hidden_states, w_tok, pos3)
    y_sorted = _grouped_mlp(x_sorted, w_sorted, W1, b1, W2, b2, sched)
    out = _sc_gather(y_sorted, pos3)
    return out
